# baseline (speedup 1.0000x reference)
import jax
import jax.numpy as jnp
from jax.experimental import pallas as pl
from jax.experimental.pallas import tpu as pltpu

_TOP_K = 8
_BM = 512
_NE = 64


def _xor_partner_rows(x, j):
    if j >= 8:
        n = x.shape[0]
        parts = [x[(b ^ 1) * j:((b ^ 1) * j) + j] for b in range(n // j)]
        return jnp.concatenate(parts, axis=0)
    row = jax.lax.broadcasted_iota(jnp.int32, x.shape, dimension=0)
    lower = (row & j) == 0
    return jnp.where(lower, pltpu.roll(x, x.shape[0] - j, 0), pltpu.roll(x, j, 0))


def _bitonic_desc_rows(x):
    n = x.shape[0]
    row = jax.lax.broadcasted_iota(jnp.int32, x.shape, dimension=0)
    for k_sz in (2, 4, 8, 16, 32, 64):
        j = k_sz // 2
        while j >= 1:
            lower = (row & j) == 0
            partner = _xor_partner_rows(x, j)
            mx = jnp.maximum(x, partner)
            mn = jnp.minimum(x, partner)
            if k_sz < n:
                desc = (row & k_sz) == 0
                take_max = jnp.logical_not(jnp.logical_xor(lower, desc))
            else:
                take_max = lower
            x = jnp.where(take_max, mx, mn)
            j //= 2
    return x


def _gating_body(x_ref, w_ref, b_ref, o_ref):
    logits = jax.lax.dot_general(
        x_ref[...], w_ref[...], (((1,), (1,)), ((), ())),
        preferred_element_type=jnp.float32)
    logits = logits + b_ref[...]
    lt = jnp.transpose(logits)
    s = _bitonic_desc_rows(lt)
    t = jnp.broadcast_to(s[_TOP_K - 1:_TOP_K, :], lt.shape)
    m = jnp.broadcast_to(s[0:1, :], lt.shape)
    e = jnp.where(lt >= t, jnp.exp(lt - m), 0.0)
    d = e[0:32] + e[32:64]
    d = d[0:16] + d[16:32]
    d = d[0:8] + d[8:16]
    d = d + pltpu.roll(d, 4, 0)
    d = d + pltpu.roll(d, 2, 0)
    d = d + pltpu.roll(d, 1, 0)
    inv = 1.0 / d
    o_ref[...] = e * jnp.concatenate([inv] * 8, axis=0)


def kernel(x, W, b):
    n_tokens, d = x.shape
    n_exp = W.shape[1]
    b2 = b.reshape(1, n_exp)
    grid = (n_tokens // _BM,)
    out_t = pl.pallas_call(
        _gating_body,
        grid=grid,
        in_specs=[
            pl.BlockSpec((_BM, d), lambda i: (i, 0)),
            pl.BlockSpec((n_exp, d), lambda i: (0, 0)),
            pl.BlockSpec((1, n_exp), lambda i: (0, 0)),
        ],
        out_specs=pl.BlockSpec((n_exp, _BM), lambda i: (0, i)),
        out_shape=jax.ShapeDtypeStruct((n_exp, n_tokens), jnp.float32),
    )(x, jnp.transpose(W), b2)
    return jnp.transpose(out_t)
